# Initial kernel scaffold; baseline (speedup 1.0000x reference)
import jax
import jax.numpy as jnp
from jax.experimental import pallas as pl


def kernel(inputs, codebook):
    raise NotImplementedError("write your pallas kernel here")


# speedup vs baseline: 1.2655x; 1.2655x over previous
import functools

import jax
import jax.numpy as jnp
from jax import lax
from jax.experimental import pallas as pl
from jax.experimental.pallas import tpu as pltpu
from jax.experimental.pallas import tpu_sc as plsc

_NUM_EMB = 8192
_EMB_DIM = 32
_COMMITMENT_COST = 0.25

_ROWS = 16384
_BLK = 256
_GRID = _ROWS // _BLK

_NC = 2
_NS = 16
_NW = _NC * _NS
_ROWS_PER_W = _ROWS // _NW
_CHUNK = 128
_NCHUNK = _ROWS_PER_W // _CHUNK


def _argmin_body(x2_ref, cb_ref, xn_ref, cn_ref, idx_ref, loss_ref):
    pid = pl.program_id(0)
    x2 = x2_ref[...]
    cb = cb_ref[...]
    xn = xn_ref[...]
    cn = cn_ref[...]
    s = lax.dot_general(x2, cb, (((1,), (1,)), ((), ())),
                        preferred_element_type=jnp.float32)
    dist = (xn - s) + cn
    m = jnp.min(dist, axis=1)
    iota = lax.broadcasted_iota(jnp.int32, dist.shape, 1)
    idx = jnp.min(jnp.where(dist == m[:, None], iota, _NUM_EMB), axis=1)
    idx_ref[...] = idx
    partial = jnp.sum(m, keepdims=True)[None, :]

    @pl.when(pid == 0)
    def _():
        loss_ref[...] = jnp.zeros((1, 1), jnp.float32)

    loss_ref[...] += partial


def _argmin_call(flat, codebook):
    xn = jnp.sum(flat ** 2, axis=1, keepdims=True)
    cn = jnp.sum(codebook ** 2, axis=1)[None, :]
    x2 = (2.0 * flat).astype(jnp.bfloat16)
    return pl.pallas_call(
        _argmin_body,
        grid=(_GRID,),
        in_specs=[
            pl.BlockSpec((_BLK, _EMB_DIM), lambda i: (i, 0)),
            pl.BlockSpec((_NUM_EMB, _EMB_DIM), lambda i: (0, 0)),
            pl.BlockSpec((_BLK, 1), lambda i: (i, 0)),
            pl.BlockSpec((1, _NUM_EMB), lambda i: (0, 0)),
        ],
        out_specs=[
            pl.BlockSpec((_BLK,), lambda i: (i,)),
            pl.BlockSpec((1, 1), lambda i: (0, 0)),
        ],
        out_shape=[
            jax.ShapeDtypeStruct((_ROWS,), jnp.int32),
            jax.ShapeDtypeStruct((1, 1), jnp.float32),
        ],
    )(x2, codebook, xn, cn)


def _gather_body(cb_hbm, idx_hbm, out_hbm, idx_v, rows_v, sem):
    wid = lax.axis_index("s") * _NC + lax.axis_index("c")
    pltpu.sync_copy(idx_hbm.at[pl.ds(wid * _NCHUNK, _NCHUNK)], idx_v)
    copies = []
    for j in range(_NCHUNK):
        copies.append(
            pltpu.async_copy(
                cb_hbm.at[idx_v.at[j]],
                rows_v.at[pl.ds(j * _CHUNK, _CHUNK)],
                sem,
            )
        )
    for c in copies:
        c.wait()
    pltpu.sync_copy(rows_v, out_hbm.at[pl.ds(wid * _ROWS_PER_W, _ROWS_PER_W)])


@functools.cache
def _make_gather():
    return pl.kernel(
        _gather_body,
        out_type=jax.ShapeDtypeStruct((_ROWS, _EMB_DIM), jnp.float32),
        mesh=plsc.VectorSubcoreMesh(core_axis_name="c", subcore_axis_name="s"),
        scratch_types=[
            pltpu.VMEM((_NCHUNK, _CHUNK), jnp.int32),
            pltpu.VMEM((_ROWS_PER_W, _EMB_DIM), jnp.float32),
            pltpu.SemaphoreType.DMA,
        ],
        compiler_params=pltpu.CompilerParams(use_tc_tiling_on_sc=False),
    )


def kernel(inputs, codebook):
    B, D, H, W = inputs.shape
    flat = jnp.transpose(inputs, (0, 2, 3, 1)).reshape(-1, D)
    indices, loss_sum = _argmin_call(flat, codebook)
    rows = _make_gather()(codebook, indices.reshape(_NW * _NCHUNK, _CHUNK))
    quantized = jnp.transpose(rows.reshape(B, H, W, D), (0, 3, 1, 2))
    vq_loss = loss_sum[0, 0] * ((1.0 + _COMMITMENT_COST) / float(_ROWS * D))
    return (quantized, vq_loss)
